# baseline (speedup 1.0000x reference)
import functools

import jax
import jax.numpy as jnp
from jax import lax
from jax.experimental import pallas as pl
from jax.experimental.pallas import tpu as pltpu
from jax.experimental.pallas import tpu_sc as plsc

N = 10000
NP = 10240
E = 320000
D = 128
H = 128
HW = H // 2
C = 40
EPS = 1e-5
INVS = 1.0 / (1.0 + EPS) ** 0.5

NC = 2
NS = 16
NW = NC * NS
PADROW = N + 8
RPT = NP // NS

DCH = 80
DNCH = 128
DEPT = E // NW
DEPTP = DCH * DNCH

SCH = 96
SNCH = 106
SEPT = E // NW
SEPTP = SCH * SNCH
NB = 2
NGRP = SNCH // NB
NV = SCH // 16

_f32 = jnp.float32


def _mesh():
    return plsc.VectorSubcoreMesh(
        core_axis_name="c", subcore_axis_name="s", num_cores=NC, num_subcores=NS
    )


@functools.cache
def _deg_kernel_build():
    return functools.partial(
        pl.kernel,
        out_type=jax.ShapeDtypeStruct((NC, NP, H), _f32),
        mesh=_mesh(),
        scratch_types=[
            pltpu.VMEM((DNCH, DCH), jnp.int32),
            pltpu.VMEM((DCH, H), _f32),
            pltpu.VMEM_SHARED((NP, H), _f32),
            pltpu.SemaphoreType.DMA,
        ],
    )(_deg_body)


def _deg_kernel(dst3, onesH, zerosH):
    return _deg_kernel_build()(dst3, onesH, zerosH)


def _deg_body(dst_hbm, ones_hbm, zeros_hbm, out_hbm, idx_v, ones_v, acc, sem):
    c = lax.axis_index("c")
    s = lax.axis_index("s")
    wid = c * NS + s
    pltpu.sync_copy(dst_hbm.at[wid], idx_v)
    pltpu.sync_copy(ones_hbm, ones_v)
    sl = pl.ds(s * RPT, RPT)
    pltpu.sync_copy(zeros_hbm.at[sl], acc.at[sl])
    plsc.subcore_barrier()

    def body(j, carry):
        pltpu.async_copy(ones_v, acc.at[idx_v.at[j]], sem, add=True)
        return carry

    lax.fori_loop(0, DNCH, body, 0)

    def drain(j, carry):
        pltpu.make_async_copy(ones_v, acc.at[idx_v.at[0]], sem).wait()
        return carry

    lax.fori_loop(0, DNCH, drain, 0)
    plsc.subcore_barrier()
    pltpu.sync_copy(acc.at[sl], out_hbm.at[c, sl])


@functools.cache
def _seg_kernel_build():
    return functools.partial(
        pl.kernel,
        out_type=jax.ShapeDtypeStruct((NC, NP, H), _f32),
        mesh=_mesh(),
        scratch_types=[
            pltpu.VMEM((SNCH, SCH), jnp.int32),
            pltpu.VMEM((SCH,), jnp.int32),
            pltpu.VMEM((SCH,), jnp.int32),
            pltpu.VMEM((SCH,), jnp.int32),
            pltpu.VMEM((SCH,), jnp.int32),
            pltpu.VMEM((SCH, H), _f32),
            pltpu.VMEM((SCH, H), _f32),
            pltpu.VMEM_SHARED((NP, H), _f32),
        ] + [pltpu.SemaphoreType.DMA] * NB,
    )(_seg_body)


def _seg_kernel(pidx3, g, zerosH):
    return _seg_kernel_build()(pidx3, g, zerosH)


def _unpack_chunk(pidx_v, sbuf, dbuf, ch):
    for k in range(NV):
        v = pidx_v[ch, pl.ds(k * 16, 16)]
        sbuf[pl.ds(k * 16, 16)] = v & 0x3FFF
        dbuf[pl.ds(k * 16, 16)] = lax.shift_right_logical(v, 14)


def _seg_body(pidx_hbm, g_hbm, zeros_hbm, out_hbm,
              pidx_v, sb0, sb1, db0, db1, rows0, rows1, acc, *sems):
    sbuf = (sb0, sb1)
    dbuf = (db0, db1)
    rows = (rows0, rows1)
    gsem = sems
    c = lax.axis_index("c")
    s = lax.axis_index("s")
    wid = c * NS + s
    pltpu.sync_copy(pidx_hbm.at[wid], pidx_v)
    sl = pl.ds(s * RPT, RPT)

    @pl.when(c == 0)
    def _():
        pltpu.sync_copy(g_hbm.at[sl], acc.at[sl])

    @pl.when(c == 1)
    def _():
        pltpu.sync_copy(zeros_hbm.at[sl], acc.at[sl])

    plsc.subcore_barrier()

    _unpack_chunk(pidx_v, sb0, db0, 0)
    pltpu.async_copy(g_hbm.at[sb0], rows0, gsem[0])

    def group(j, carry):
        for b in range(NB):
            ch = j * NB + b
            nb = (b + 1) % NB
            pltpu.make_async_copy(g_hbm.at[sbuf[b]], rows[b],
                                  gsem[b]).wait()

            @pl.when(ch + 1 < SNCH)
            def _():
                _unpack_chunk(pidx_v, sbuf[nb], dbuf[nb], ch + 1)
                pltpu.async_copy(g_hbm.at[sbuf[nb]], rows[nb], gsem[nb])

            pltpu.sync_copy(rows[b], acc.at[dbuf[b]], add=True)
        return carry

    lax.fori_loop(0, NGRP, group, 0)
    plsc.subcore_barrier()
    pltpu.sync_copy(acc.at[sl], out_hbm.at[c, sl])


def _pre_body(degp_ref, x_ref, w_ref, g_ref, dinv_ref):
    deg = degp_ref[0, :, 0] + degp_ref[1, :, 0] + 1.0
    dinv = lax.rsqrt(deg)[:, None]
    r = jnp.dot(x_ref[...], w_ref[...], preferred_element_type=_f32)
    g_ref[...] = r * dinv
    dinv_ref[...] = dinv


def _pre_call(degp, x, w):
    return pl.pallas_call(
        _pre_body,
        out_shape=[
            jax.ShapeDtypeStruct((NP, H), _f32),
            jax.ShapeDtypeStruct((NP, 1), _f32),
        ],
    )(degp, x, w)


def _stage_body(sp_ref, dinv_ref, cb_ref, bg_ref, bb_ref, w_ref, g_ref):
    dinv = dinv_ref[...]
    t = (sp_ref[0] + sp_ref[1]) * dinv + cb_ref[...]
    h = jnp.maximum(t * (INVS * bg_ref[...]) + bb_ref[...], 0.0)
    g_ref[...] = jnp.dot(h, w_ref[...], preferred_element_type=_f32) * dinv


def _stage_call(sp, dinv, cb, bg, bb, w):
    return pl.pallas_call(
        _stage_body,
        out_shape=jax.ShapeDtypeStruct((NP, H), _f32),
    )(sp, dinv, cb, bg, bb, w)


def _log_softmax(z):
    m = jnp.max(z, axis=-1, keepdims=True)
    e = jnp.exp(z - m)
    return z - m - jnp.log(jnp.sum(e, axis=-1, keepdims=True))


def _heads_body(sp_ref, dinv_ref, cb_ref, bg_ref, bb_ref,
                fcgw0_ref, fcgw1_ref, fcgw2_ref, fcgb_ref,
                bngg_ref, bngb_ref,
                fclw_ref, fclb_ref, bnlg_ref, bnlb_ref,
                outgw_ref, outgb_ref, outlw_ref, outlb_ref,
                og_ref, ol0_ref, ol1_ref, ol2_ref):
    dinv = dinv_ref[...]
    t = (sp_ref[0] + sp_ref[1]) * dinv + cb_ref[...]
    h = jnp.maximum(t * (INVS * bg_ref[...]) + bb_ref[...], 0.0)

    def bnrelu(z, g, b):
        return jnp.maximum(z * (INVS * g) + b, 0.0)

    xg0 = jnp.dot(h, fcgw0_ref[...], preferred_element_type=_f32) + fcgb_ref[0]
    xg0 = bnrelu(xg0, bngg_ref[0], bngb_ref[0])
    w1 = fcgw1_ref[...]
    xg1 = (jnp.dot(xg0, w1[:H], preferred_element_type=_f32)
           + jnp.dot(h, w1[H:], preferred_element_type=_f32) + fcgb_ref[1])
    xg1 = bnrelu(xg1, bngg_ref[1], bngb_ref[1])
    w2 = fcgw2_ref[...]
    xg2 = (jnp.dot(xg1, w2[:H], preferred_element_type=_f32)
           + jnp.dot(h, w2[H:], preferred_element_type=_f32) + fcgb_ref[2])
    xg2 = bnrelu(xg2, bngg_ref[2], bngb_ref[2])

    og = jnp.dot(xg2, outgw_ref[...], preferred_element_type=_f32) + outgb_ref[...]
    og_ref[...] = _log_softmax(og)

    for i, (xg, ol_ref) in enumerate(((xg0, ol0_ref), (xg1, ol1_ref), (xg2, ol2_ref))):
        hl = jnp.dot(xg, fclw_ref[i], preferred_element_type=_f32) + fclb_ref[i]
        hl = bnrelu(hl, bnlg_ref[i], bnlb_ref[i])
        ol = jnp.dot(hl, outlw_ref[i], preferred_element_type=_f32) + outlb_ref[i]
        ol_ref[...] = _log_softmax(ol)


def _heads_call(sp, dinv, cb, bg, bb, p):
    out_shape = [jax.ShapeDtypeStruct((NP, C), _f32) for _ in range(4)]
    return pl.pallas_call(_heads_body, out_shape=out_shape)(
        sp, dinv, cb, bg, bb,
        p["fcg_W"][0], p["fcg_W"][1], p["fcg_W"][2], jnp.stack(p["fcg_b"]),
        jnp.stack(p["bng_g"]), jnp.stack(p["bng_b"]),
        jnp.stack(p["fcl_W"]), jnp.stack(p["fcl_b"]),
        jnp.stack(p["bnl_g"]), jnp.stack(p["bnl_b"]),
        p["outg_W"], p["outg_b"], jnp.stack(p["outl_W"]), jnp.stack(p["outl_b"]),
    )


def kernel(x, adj_t, params):
    src, dst = adj_t[0], adj_t[1]
    dpad = ((0, 0), (0, DEPTP - DEPT))
    dst3d = jnp.pad(dst.reshape(NW, DEPT), dpad,
                    constant_values=PADROW).reshape(NW, DNCH, DCH)
    spad = ((0, 0), (0, SEPTP - SEPT))
    packed = src + (dst << 14)
    pidx3 = jnp.pad(packed.reshape(NW, SEPT), spad,
                    constant_values=PADROW + (PADROW << 14)
                    ).reshape(NW, SNCH, SCH)

    onesH = jnp.ones((DCH, H), _f32)
    zerosH = jnp.zeros((NP, H), _f32)
    xp = jnp.pad(x, ((0, NP - N), (0, 0)))

    degp = _deg_kernel(dst3d, onesH, zerosH)
    g, dinv = _pre_call(degp, xp, params["conv_W"][0])
    for i in range(2):
        sp = _seg_kernel(pidx3, g, zerosH)
        g = _stage_call(sp, dinv, params["conv_b"][i], params["bn_g"][i],
                        params["bn_b"][i], params["conv_W"][i + 1])
    sp = _seg_kernel(pidx3, g, zerosH)
    og, ol0, ol1, ol2 = _heads_call(sp, dinv, params["conv_b"][2],
                                    params["bn_g"][2], params["bn_b"][2], params)
    return og[:N], ol0[:N], ol1[:N], ol2[:N]


# speedup vs baseline: 1.0007x; 1.0007x over previous
import functools

import jax
import jax.numpy as jnp
from jax import lax
from jax.experimental import pallas as pl
from jax.experimental.pallas import tpu as pltpu
from jax.experimental.pallas import tpu_sc as plsc

N = 10000
NP = 10240
E = 320000
D = 128
H = 128
HW = H // 2
C = 40
EPS = 1e-5
INVS = 1.0 / (1.0 + EPS) ** 0.5

NC = 2
NS = 16
NW = NC * NS
PADROW = N + 8
RPT = NP // NS

DCH = 80
DNCH = 128
DEPT = E // NW
DEPTP = DCH * DNCH

SCH = 96
SNCH = 106
SEPT = E // NW
SEPTP = SCH * SNCH
NB = 2
NGRP = SNCH // NB
NV = SCH // 16

_f32 = jnp.float32


def _mesh():
    return plsc.VectorSubcoreMesh(
        core_axis_name="c", subcore_axis_name="s", num_cores=NC, num_subcores=NS
    )


@functools.cache
def _deg_kernel_build():
    return functools.partial(
        pl.kernel,
        out_type=jax.ShapeDtypeStruct((NC, NP, H), _f32),
        mesh=_mesh(),
        scratch_types=[
            pltpu.VMEM((DNCH, DCH), jnp.int32),
            pltpu.VMEM((DCH, H), _f32),
            pltpu.VMEM_SHARED((NP, H), _f32),
            pltpu.SemaphoreType.DMA,
        ],
    )(_deg_body)


def _deg_kernel(dst3, onesH, zerosH):
    return _deg_kernel_build()(dst3, onesH, zerosH)


def _deg_body(dst_hbm, ones_hbm, zeros_hbm, out_hbm, idx_v, ones_v, acc, sem):
    c = lax.axis_index("c")
    s = lax.axis_index("s")
    wid = c * NS + s
    pltpu.sync_copy(dst_hbm.at[wid], idx_v)
    pltpu.sync_copy(ones_hbm, ones_v)
    sl = pl.ds(s * RPT, RPT)
    pltpu.sync_copy(zeros_hbm.at[sl], acc.at[sl])
    plsc.subcore_barrier()

    def body(j, carry):
        pltpu.async_copy(ones_v, acc.at[idx_v.at[j]], sem, add=True)
        return carry

    lax.fori_loop(0, DNCH, body, 0)

    def drain(j, carry):
        pltpu.make_async_copy(ones_v, acc.at[idx_v.at[0]], sem).wait()
        return carry

    lax.fori_loop(0, DNCH, drain, 0)
    plsc.subcore_barrier()
    pltpu.sync_copy(acc.at[sl], out_hbm.at[c, sl])


@functools.cache
def _seg_kernel_build():
    return functools.partial(
        pl.kernel,
        out_type=jax.ShapeDtypeStruct((NC, NP, H), _f32),
        mesh=_mesh(),
        scratch_types=[
            pltpu.VMEM((SNCH, SCH), jnp.int32),
            pltpu.VMEM((SCH,), jnp.int32),
            pltpu.VMEM((SCH,), jnp.int32),
            pltpu.VMEM((SCH,), jnp.int32),
            pltpu.VMEM((SCH,), jnp.int32),
            pltpu.VMEM((SCH, H), _f32),
            pltpu.VMEM((SCH, H), _f32),
            pltpu.VMEM_SHARED((NP, H), _f32),
        ] + [pltpu.SemaphoreType.DMA] * NB,
    )(_seg_body)


def _seg_kernel(pidx3, g, zerosH):
    return _seg_kernel_build()(pidx3, g, zerosH)


def _unpack_chunk(pidx_v, sbuf, dbuf, ch):
    for k in range(NV):
        v = pidx_v[ch, pl.ds(k * 16, 16)]
        sbuf[pl.ds(k * 16, 16)] = v & 0x3FFF
        dbuf[pl.ds(k * 16, 16)] = lax.shift_right_logical(v, 14)


def _seg_body(pidx_hbm, g_hbm, zeros_hbm, out_hbm,
              pidx_v, sb0, sb1, db0, db1, rows0, rows1, acc, *sems):
    sbuf = (sb0, sb1)
    dbuf = (db0, db1)
    rows = (rows0, rows1)
    gsem = sems
    c = lax.axis_index("c")
    s = lax.axis_index("s")
    wid = c * NS + s
    pltpu.sync_copy(pidx_hbm.at[wid], pidx_v)
    sl = pl.ds(s * RPT, RPT)

    @pl.when(c == 0)
    def _():
        pltpu.sync_copy(g_hbm.at[sl], acc.at[sl])

    @pl.when(c == 1)
    def _():
        pltpu.sync_copy(zeros_hbm.at[sl], acc.at[sl])

    plsc.subcore_barrier()

    _unpack_chunk(pidx_v, sb0, db0, 0)
    pltpu.async_copy(g_hbm.at[sb0], rows0, gsem[0])

    def group(j, carry):
        for b in range(NB):
            ch = j * NB + b
            nb = (b + 1) % NB
            pltpu.make_async_copy(g_hbm.at[sbuf[b]], rows[b],
                                  gsem[b]).wait()
            _unpack_chunk(pidx_v, sbuf[nb], dbuf[nb], ch + 1)
            pltpu.async_copy(g_hbm.at[sbuf[nb]], rows[nb], gsem[nb])
            pltpu.sync_copy(rows[b], acc.at[dbuf[b]], add=True)
        return carry

    lax.fori_loop(0, NGRP - 1, group, 0)
    pltpu.make_async_copy(g_hbm.at[sb0], rows0, gsem[0]).wait()
    _unpack_chunk(pidx_v, sb1, db1, SNCH - 1)
    pltpu.async_copy(g_hbm.at[sb1], rows1, gsem[1])
    pltpu.sync_copy(rows0, acc.at[db0], add=True)
    pltpu.make_async_copy(g_hbm.at[sb1], rows1, gsem[1]).wait()
    pltpu.sync_copy(rows1, acc.at[db1], add=True)
    plsc.subcore_barrier()
    pltpu.sync_copy(acc.at[sl], out_hbm.at[c, sl])


def _pre_body(degp_ref, x_ref, w_ref, g_ref, dinv_ref):
    deg = degp_ref[0, :, 0] + degp_ref[1, :, 0] + 1.0
    dinv = lax.rsqrt(deg)[:, None]
    r = jnp.dot(x_ref[...], w_ref[...], preferred_element_type=_f32)
    g_ref[...] = r * dinv
    dinv_ref[...] = dinv


def _pre_call(degp, x, w):
    return pl.pallas_call(
        _pre_body,
        out_shape=[
            jax.ShapeDtypeStruct((NP, H), _f32),
            jax.ShapeDtypeStruct((NP, 1), _f32),
        ],
    )(degp, x, w)


def _stage_body(sp_ref, dinv_ref, cb_ref, bg_ref, bb_ref, w_ref, g_ref):
    dinv = dinv_ref[...]
    t = (sp_ref[0] + sp_ref[1]) * dinv + cb_ref[...]
    h = jnp.maximum(t * (INVS * bg_ref[...]) + bb_ref[...], 0.0)
    g_ref[...] = jnp.dot(h, w_ref[...], preferred_element_type=_f32) * dinv


def _stage_call(sp, dinv, cb, bg, bb, w):
    return pl.pallas_call(
        _stage_body,
        out_shape=jax.ShapeDtypeStruct((NP, H), _f32),
    )(sp, dinv, cb, bg, bb, w)


def _log_softmax(z):
    m = jnp.max(z, axis=-1, keepdims=True)
    e = jnp.exp(z - m)
    return z - m - jnp.log(jnp.sum(e, axis=-1, keepdims=True))


def _heads_body(sp_ref, dinv_ref, cb_ref, bg_ref, bb_ref,
                fcgw0_ref, fcgw1_ref, fcgw2_ref, fcgb_ref,
                bngg_ref, bngb_ref,
                fclw_ref, fclb_ref, bnlg_ref, bnlb_ref,
                outgw_ref, outgb_ref, outlw_ref, outlb_ref,
                og_ref, ol0_ref, ol1_ref, ol2_ref):
    dinv = dinv_ref[...]
    t = (sp_ref[0] + sp_ref[1]) * dinv + cb_ref[...]
    h = jnp.maximum(t * (INVS * bg_ref[...]) + bb_ref[...], 0.0)

    def bnrelu(z, g, b):
        return jnp.maximum(z * (INVS * g) + b, 0.0)

    xg0 = jnp.dot(h, fcgw0_ref[...], preferred_element_type=_f32) + fcgb_ref[0]
    xg0 = bnrelu(xg0, bngg_ref[0], bngb_ref[0])
    w1 = fcgw1_ref[...]
    xg1 = (jnp.dot(xg0, w1[:H], preferred_element_type=_f32)
           + jnp.dot(h, w1[H:], preferred_element_type=_f32) + fcgb_ref[1])
    xg1 = bnrelu(xg1, bngg_ref[1], bngb_ref[1])
    w2 = fcgw2_ref[...]
    xg2 = (jnp.dot(xg1, w2[:H], preferred_element_type=_f32)
           + jnp.dot(h, w2[H:], preferred_element_type=_f32) + fcgb_ref[2])
    xg2 = bnrelu(xg2, bngg_ref[2], bngb_ref[2])

    og = jnp.dot(xg2, outgw_ref[...], preferred_element_type=_f32) + outgb_ref[...]
    og_ref[...] = _log_softmax(og)

    for i, (xg, ol_ref) in enumerate(((xg0, ol0_ref), (xg1, ol1_ref), (xg2, ol2_ref))):
        hl = jnp.dot(xg, fclw_ref[i], preferred_element_type=_f32) + fclb_ref[i]
        hl = bnrelu(hl, bnlg_ref[i], bnlb_ref[i])
        ol = jnp.dot(hl, outlw_ref[i], preferred_element_type=_f32) + outlb_ref[i]
        ol_ref[...] = _log_softmax(ol)


def _heads_call(sp, dinv, cb, bg, bb, p):
    out_shape = [jax.ShapeDtypeStruct((NP, C), _f32) for _ in range(4)]
    return pl.pallas_call(_heads_body, out_shape=out_shape)(
        sp, dinv, cb, bg, bb,
        p["fcg_W"][0], p["fcg_W"][1], p["fcg_W"][2], jnp.stack(p["fcg_b"]),
        jnp.stack(p["bng_g"]), jnp.stack(p["bng_b"]),
        jnp.stack(p["fcl_W"]), jnp.stack(p["fcl_b"]),
        jnp.stack(p["bnl_g"]), jnp.stack(p["bnl_b"]),
        p["outg_W"], p["outg_b"], jnp.stack(p["outl_W"]), jnp.stack(p["outl_b"]),
    )


def kernel(x, adj_t, params):
    src, dst = adj_t[0], adj_t[1]
    dpad = ((0, 0), (0, DEPTP - DEPT))
    dst3d = jnp.pad(dst.reshape(NW, DEPT), dpad,
                    constant_values=PADROW).reshape(NW, DNCH, DCH)
    spad = ((0, 0), (0, SEPTP - SEPT))
    packed = src + (dst << 14)
    pidx3 = jnp.pad(packed.reshape(NW, SEPT), spad,
                    constant_values=PADROW + (PADROW << 14)
                    ).reshape(NW, SNCH, SCH)

    onesH = jnp.ones((DCH, H), _f32)
    zerosH = jnp.zeros((NP, H), _f32)
    xp = jnp.pad(x, ((0, NP - N), (0, 0)))

    degp = _deg_kernel(dst3d, onesH, zerosH)
    g, dinv = _pre_call(degp, xp, params["conv_W"][0])
    for i in range(2):
        sp = _seg_kernel(pidx3, g, zerosH)
        g = _stage_call(sp, dinv, params["conv_b"][i], params["bn_g"][i],
                        params["bn_b"][i], params["conv_W"][i + 1])
    sp = _seg_kernel(pidx3, g, zerosH)
    og, ol0, ol1, ol2 = _heads_call(sp, dinv, params["conv_b"][2],
                                    params["bn_g"][2], params["bn_b"][2], params)
    return og[:N], ol0[:N], ol1[:N], ol2[:N]


# speedup vs baseline: 1.5425x; 1.5414x over previous
import functools

import jax
import jax.numpy as jnp
from jax import lax
from jax.experimental import pallas as pl
from jax.experimental.pallas import tpu as pltpu
from jax.experimental.pallas import tpu_sc as plsc

N = 10000
NP = 10240
E = 320000
D = 128
H = 128
C = 40
EPS = 1e-5
INVS = 1.0 / (1.0 + EPS) ** 0.5

NC = 2
NS = 16
NW = NC * NS
EPT = E // NW
CHUNK = 80
NCH = EPT // CHUNK
RPT = NP // NS

_f32 = jnp.float32


@functools.cache
def _deg_kernel_build():
    mesh = plsc.VectorSubcoreMesh(
        core_axis_name="c", subcore_axis_name="s", num_cores=NC, num_subcores=NS
    )
    return functools.partial(
        pl.kernel,
        out_type=jax.ShapeDtypeStruct((NC, NP, H), _f32),
        mesh=mesh,
        scratch_types=[
            pltpu.VMEM((NCH, CHUNK), jnp.int32),
            pltpu.VMEM((CHUNK, H), _f32),
            pltpu.VMEM_SHARED((NP, H), _f32),
        ],
    )(_deg_body)


def _deg_kernel(dst3, onesH, zerosH):
    return _deg_kernel_build()(dst3, onesH, zerosH)


def _deg_body(dst_hbm, ones_hbm, zeros_hbm, out_hbm, idx_v, ones_v, acc):
    c = lax.axis_index("c")
    s = lax.axis_index("s")
    wid = c * NS + s
    pltpu.sync_copy(dst_hbm.at[wid], idx_v)
    pltpu.sync_copy(ones_hbm, ones_v)
    sl = pl.ds(s * RPT, RPT)
    pltpu.sync_copy(zeros_hbm.at[sl], acc.at[sl])
    plsc.subcore_barrier()

    def body(j, carry):
        pltpu.sync_copy(ones_v, acc.at[idx_v.at[j]], add=True)
        return carry

    lax.fori_loop(0, NCH, body, 0)
    plsc.subcore_barrier()
    pltpu.sync_copy(acc.at[sl], out_hbm.at[c, sl])


@functools.cache
def _seg_kernel_build():
    mesh = plsc.VectorSubcoreMesh(
        core_axis_name="c", subcore_axis_name="s", num_cores=NC, num_subcores=NS
    )
    return functools.partial(
        pl.kernel,
        out_type=jax.ShapeDtypeStruct((NC, NP, H), _f32),
        mesh=mesh,
        scratch_types=[
            pltpu.VMEM((NCH, CHUNK), jnp.int32),
            pltpu.VMEM((NCH, CHUNK), jnp.int32),
            pltpu.VMEM((CHUNK, H), _f32),
            pltpu.VMEM_SHARED((NP, H), _f32),
            pltpu.SemaphoreType.DMA,
        ],
    )(_seg_body)


def _seg_kernel(src3, dst3, g, zerosH):
    return _seg_kernel_build()(src3, dst3, g, zerosH)


def _seg_body(src_hbm, dst_hbm, g_hbm, zeros_hbm, out_hbm,
              src_v, dst_v, rows_v, acc, sem):
    c = lax.axis_index("c")
    s = lax.axis_index("s")
    wid = c * NS + s
    pltpu.sync_copy(src_hbm.at[wid], src_v)
    pltpu.sync_copy(dst_hbm.at[wid], dst_v)
    sl = pl.ds(s * RPT, RPT)

    @pl.when(c == 0)
    def _():
        pltpu.sync_copy(g_hbm.at[sl], acc.at[sl])

    @pl.when(c == 1)
    def _():
        pltpu.sync_copy(zeros_hbm.at[sl], acc.at[sl])

    plsc.subcore_barrier()

    def body(j, carry):
        pltpu.async_copy(g_hbm.at[src_v.at[j]], rows_v, sem).wait()
        pltpu.sync_copy(rows_v, acc.at[dst_v.at[j]], add=True)
        return carry

    lax.fori_loop(0, NCH, body, 0)
    plsc.subcore_barrier()
    pltpu.sync_copy(acc.at[sl], out_hbm.at[c, sl])


def _pre_body(degp_ref, x_ref, w_ref, g_ref, dinv_ref):
    deg = degp_ref[0, :, 0] + degp_ref[1, :, 0] + 1.0
    dinv = lax.rsqrt(deg)[:, None]
    r = jnp.dot(x_ref[...], w_ref[...], preferred_element_type=_f32)
    g_ref[...] = r * dinv
    dinv_ref[...] = dinv


def _pre_call(degp, x, w):
    return pl.pallas_call(
        _pre_body,
        out_shape=[
            jax.ShapeDtypeStruct((NP, H), _f32),
            jax.ShapeDtypeStruct((NP, 1), _f32),
        ],
    )(degp, x, w)


def _stage_body(sp_ref, dinv_ref, cb_ref, bg_ref, bb_ref, w_ref, g_ref):
    dinv = dinv_ref[...]
    t = (sp_ref[0] + sp_ref[1]) * dinv + cb_ref[...]
    h = jnp.maximum(t * (INVS * bg_ref[...]) + bb_ref[...], 0.0)
    g_ref[...] = jnp.dot(h, w_ref[...], preferred_element_type=_f32) * dinv


def _stage_call(sp, dinv, cb, bg, bb, w):
    return pl.pallas_call(
        _stage_body,
        out_shape=jax.ShapeDtypeStruct((NP, H), _f32),
    )(sp, dinv, cb, bg, bb, w)


def _log_softmax(z):
    m = jnp.max(z, axis=-1, keepdims=True)
    e = jnp.exp(z - m)
    return z - m - jnp.log(jnp.sum(e, axis=-1, keepdims=True))


def _heads_body(sp_ref, dinv_ref, cb_ref, bg_ref, bb_ref,
                fcgw0_ref, fcgw1_ref, fcgw2_ref, fcgb_ref,
                bngg_ref, bngb_ref,
                fclw_ref, fclb_ref, bnlg_ref, bnlb_ref,
                outgw_ref, outgb_ref, outlw_ref, outlb_ref,
                og_ref, ol0_ref, ol1_ref, ol2_ref):
    dinv = dinv_ref[...]
    t = (sp_ref[0] + sp_ref[1]) * dinv + cb_ref[...]
    h = jnp.maximum(t * (INVS * bg_ref[...]) + bb_ref[...], 0.0)

    def bnrelu(z, g, b):
        return jnp.maximum(z * (INVS * g) + b, 0.0)

    xg0 = jnp.dot(h, fcgw0_ref[...], preferred_element_type=_f32) + fcgb_ref[0]
    xg0 = bnrelu(xg0, bngg_ref[0], bngb_ref[0])
    w1 = fcgw1_ref[...]
    xg1 = (jnp.dot(xg0, w1[:H], preferred_element_type=_f32)
           + jnp.dot(h, w1[H:], preferred_element_type=_f32) + fcgb_ref[1])
    xg1 = bnrelu(xg1, bngg_ref[1], bngb_ref[1])
    w2 = fcgw2_ref[...]
    xg2 = (jnp.dot(xg1, w2[:H], preferred_element_type=_f32)
           + jnp.dot(h, w2[H:], preferred_element_type=_f32) + fcgb_ref[2])
    xg2 = bnrelu(xg2, bngg_ref[2], bngb_ref[2])

    og = jnp.dot(xg2, outgw_ref[...], preferred_element_type=_f32) + outgb_ref[...]
    og_ref[...] = _log_softmax(og)

    for i, (xg, ol_ref) in enumerate(((xg0, ol0_ref), (xg1, ol1_ref), (xg2, ol2_ref))):
        hl = jnp.dot(xg, fclw_ref[i], preferred_element_type=_f32) + fclb_ref[i]
        hl = bnrelu(hl, bnlg_ref[i], bnlb_ref[i])
        ol = jnp.dot(hl, outlw_ref[i], preferred_element_type=_f32) + outlb_ref[i]
        ol_ref[...] = _log_softmax(ol)


def _heads_call(sp, dinv, cb, bg, bb, p):
    out_shape = [jax.ShapeDtypeStruct((NP, C), _f32) for _ in range(4)]
    return pl.pallas_call(_heads_body, out_shape=out_shape)(
        sp, dinv, cb, bg, bb,
        p["fcg_W"][0], p["fcg_W"][1], p["fcg_W"][2], jnp.stack(p["fcg_b"]),
        jnp.stack(p["bng_g"]), jnp.stack(p["bng_b"]),
        jnp.stack(p["fcl_W"]), jnp.stack(p["fcl_b"]),
        jnp.stack(p["bnl_g"]), jnp.stack(p["bnl_b"]),
        p["outg_W"], p["outg_b"], jnp.stack(p["outl_W"]), jnp.stack(p["outl_b"]),
    )


def kernel(x, adj_t, params):
    src3 = adj_t[0].reshape(NW, NCH, CHUNK)
    dst3 = adj_t[1].reshape(NW, NCH, CHUNK)
    onesH = jnp.ones((CHUNK, H), _f32)
    zerosH = jnp.zeros((NP, H), _f32)
    xp = jnp.pad(x, ((0, NP - N), (0, 0)))

    degp = _deg_kernel(dst3, onesH, zerosH)
    g, dinv = _pre_call(degp, xp, params["conv_W"][0])
    for i in range(2):
        sp = _seg_kernel(src3, dst3, g, zerosH)
        g = _stage_call(sp, dinv, params["conv_b"][i], params["bn_g"][i],
                        params["bn_b"][i], params["conv_W"][i + 1])
    sp = _seg_kernel(src3, dst3, g, zerosH)
    og, ol0, ol1, ol2 = _heads_call(sp, dinv, params["conv_b"][2],
                                    params["bn_g"][2], params["bn_b"][2], params)
    return og[:N], ol0[:N], ol1[:N], ol2[:N]


# speedup vs baseline: 1.5431x; 1.0004x over previous
import functools

import jax
import jax.numpy as jnp
from jax import lax
from jax.experimental import pallas as pl
from jax.experimental.pallas import tpu as pltpu
from jax.experimental.pallas import tpu_sc as plsc

N = 10000
NP = 10240
E = 320000
D = 128
H = 128
C = 40
EPS = 1e-5
INVS = 1.0 / (1.0 + EPS) ** 0.5

NC = 2
NS = 16
NW = NC * NS
EPT = E // NW
CHUNK = 80
NCH = EPT // CHUNK
RPT = NP // NS

_f32 = jnp.float32


@functools.cache
def _deg_kernel_build():
    mesh = plsc.VectorSubcoreMesh(
        core_axis_name="c", subcore_axis_name="s", num_cores=NC, num_subcores=NS
    )
    return functools.partial(
        pl.kernel,
        out_type=jax.ShapeDtypeStruct((NC, NP, H), _f32),
        mesh=mesh,
        scratch_types=[
            pltpu.VMEM((NCH, CHUNK), jnp.int32),
            pltpu.VMEM((CHUNK, H), _f32),
            pltpu.VMEM_SHARED((NP, H), _f32),
            pltpu.SemaphoreType.DMA,
        ],
    )(_deg_body)


def _deg_kernel(dst3, onesH, zerosH):
    return _deg_kernel_build()(dst3, onesH, zerosH)


def _deg_body(dst_hbm, ones_hbm, zeros_hbm, out_hbm, idx_v, ones_v, acc, sem):
    c = lax.axis_index("c")
    s = lax.axis_index("s")
    wid = c * NS + s
    pltpu.sync_copy(dst_hbm.at[wid], idx_v)
    pltpu.sync_copy(ones_hbm, ones_v)
    sl = pl.ds(s * RPT, RPT)
    pltpu.sync_copy(zeros_hbm.at[sl], acc.at[sl])
    plsc.subcore_barrier()

    def body(j, carry):
        pltpu.async_copy(ones_v, acc.at[idx_v.at[j]], sem, add=True)
        return carry

    lax.fori_loop(0, NCH, body, 0)

    def drain(j, carry):
        pltpu.make_async_copy(ones_v, acc.at[idx_v.at[0]], sem).wait()
        return carry

    lax.fori_loop(0, NCH, drain, 0)
    plsc.subcore_barrier()
    pltpu.sync_copy(acc.at[sl], out_hbm.at[c, sl])


@functools.cache
def _seg_kernel_build():
    mesh = plsc.VectorSubcoreMesh(
        core_axis_name="c", subcore_axis_name="s", num_cores=NC, num_subcores=NS
    )
    return functools.partial(
        pl.kernel,
        out_type=jax.ShapeDtypeStruct((NC, NP, H), _f32),
        mesh=mesh,
        scratch_types=[
            pltpu.VMEM((NCH, CHUNK), jnp.int32),
            pltpu.VMEM((NCH, CHUNK), jnp.int32),
            pltpu.VMEM((CHUNK, H), _f32),
            pltpu.VMEM_SHARED((NP, H), _f32),
            pltpu.SemaphoreType.DMA,
        ],
    )(_seg_body)


def _seg_kernel(src3, dst3, g, zerosH):
    return _seg_kernel_build()(src3, dst3, g, zerosH)


def _seg_body(src_hbm, dst_hbm, g_hbm, zeros_hbm, out_hbm,
              src_v, dst_v, rows_v, acc, sem):
    c = lax.axis_index("c")
    s = lax.axis_index("s")
    wid = c * NS + s
    pltpu.sync_copy(src_hbm.at[wid], src_v)
    pltpu.sync_copy(dst_hbm.at[wid], dst_v)
    sl = pl.ds(s * RPT, RPT)

    @pl.when(c == 0)
    def _():
        pltpu.sync_copy(g_hbm.at[sl], acc.at[sl])

    @pl.when(c == 1)
    def _():
        pltpu.sync_copy(zeros_hbm.at[sl], acc.at[sl])

    plsc.subcore_barrier()

    def body(j, carry):
        pltpu.async_copy(g_hbm.at[src_v.at[j]], rows_v, sem).wait()
        pltpu.sync_copy(rows_v, acc.at[dst_v.at[j]], add=True)
        return carry

    lax.fori_loop(0, NCH, body, 0)
    plsc.subcore_barrier()
    pltpu.sync_copy(acc.at[sl], out_hbm.at[c, sl])


def _pre_body(degp_ref, x_ref, w_ref, g_ref, dinv_ref):
    deg = degp_ref[0, :, 0] + degp_ref[1, :, 0] + 1.0
    dinv = lax.rsqrt(deg)[:, None]
    r = jnp.dot(x_ref[...], w_ref[...], preferred_element_type=_f32)
    g_ref[...] = r * dinv
    dinv_ref[...] = dinv


def _pre_call(degp, x, w):
    return pl.pallas_call(
        _pre_body,
        out_shape=[
            jax.ShapeDtypeStruct((NP, H), _f32),
            jax.ShapeDtypeStruct((NP, 1), _f32),
        ],
    )(degp, x, w)


def _stage_body(sp_ref, dinv_ref, cb_ref, bg_ref, bb_ref, w_ref, g_ref):
    dinv = dinv_ref[...]
    t = (sp_ref[0] + sp_ref[1]) * dinv + cb_ref[...]
    h = jnp.maximum(t * (INVS * bg_ref[...]) + bb_ref[...], 0.0)
    g_ref[...] = jnp.dot(h, w_ref[...], preferred_element_type=_f32) * dinv


def _stage_call(sp, dinv, cb, bg, bb, w):
    return pl.pallas_call(
        _stage_body,
        out_shape=jax.ShapeDtypeStruct((NP, H), _f32),
    )(sp, dinv, cb, bg, bb, w)


def _log_softmax(z):
    m = jnp.max(z, axis=-1, keepdims=True)
    e = jnp.exp(z - m)
    return z - m - jnp.log(jnp.sum(e, axis=-1, keepdims=True))


def _heads_body(sp_ref, dinv_ref, cb_ref, bg_ref, bb_ref,
                fcgw0_ref, fcgw1_ref, fcgw2_ref, fcgb_ref,
                bngg_ref, bngb_ref,
                fclw_ref, fclb_ref, bnlg_ref, bnlb_ref,
                outgw_ref, outgb_ref, outlw_ref, outlb_ref,
                og_ref, ol0_ref, ol1_ref, ol2_ref):
    dinv = dinv_ref[...]
    t = (sp_ref[0] + sp_ref[1]) * dinv + cb_ref[...]
    h = jnp.maximum(t * (INVS * bg_ref[...]) + bb_ref[...], 0.0)

    def bnrelu(z, g, b):
        return jnp.maximum(z * (INVS * g) + b, 0.0)

    xg0 = jnp.dot(h, fcgw0_ref[...], preferred_element_type=_f32) + fcgb_ref[0]
    xg0 = bnrelu(xg0, bngg_ref[0], bngb_ref[0])
    w1 = fcgw1_ref[...]
    xg1 = (jnp.dot(xg0, w1[:H], preferred_element_type=_f32)
           + jnp.dot(h, w1[H:], preferred_element_type=_f32) + fcgb_ref[1])
    xg1 = bnrelu(xg1, bngg_ref[1], bngb_ref[1])
    w2 = fcgw2_ref[...]
    xg2 = (jnp.dot(xg1, w2[:H], preferred_element_type=_f32)
           + jnp.dot(h, w2[H:], preferred_element_type=_f32) + fcgb_ref[2])
    xg2 = bnrelu(xg2, bngg_ref[2], bngb_ref[2])

    og = jnp.dot(xg2, outgw_ref[...], preferred_element_type=_f32) + outgb_ref[...]
    og_ref[...] = _log_softmax(og)

    for i, (xg, ol_ref) in enumerate(((xg0, ol0_ref), (xg1, ol1_ref), (xg2, ol2_ref))):
        hl = jnp.dot(xg, fclw_ref[i], preferred_element_type=_f32) + fclb_ref[i]
        hl = bnrelu(hl, bnlg_ref[i], bnlb_ref[i])
        ol = jnp.dot(hl, outlw_ref[i], preferred_element_type=_f32) + outlb_ref[i]
        ol_ref[...] = _log_softmax(ol)


def _heads_call(sp, dinv, cb, bg, bb, p):
    out_shape = [jax.ShapeDtypeStruct((NP, C), _f32) for _ in range(4)]
    return pl.pallas_call(_heads_body, out_shape=out_shape)(
        sp, dinv, cb, bg, bb,
        p["fcg_W"][0], p["fcg_W"][1], p["fcg_W"][2], jnp.stack(p["fcg_b"]),
        jnp.stack(p["bng_g"]), jnp.stack(p["bng_b"]),
        jnp.stack(p["fcl_W"]), jnp.stack(p["fcl_b"]),
        jnp.stack(p["bnl_g"]), jnp.stack(p["bnl_b"]),
        p["outg_W"], p["outg_b"], jnp.stack(p["outl_W"]), jnp.stack(p["outl_b"]),
    )


def kernel(x, adj_t, params):
    src3 = adj_t[0].reshape(NW, NCH, CHUNK)
    dst3 = adj_t[1].reshape(NW, NCH, CHUNK)
    onesH = jnp.ones((CHUNK, H), _f32)
    zerosH = jnp.zeros((NP, H), _f32)
    xp = jnp.pad(x, ((0, NP - N), (0, 0)))

    degp = _deg_kernel(dst3, onesH, zerosH)
    g, dinv = _pre_call(degp, xp, params["conv_W"][0])
    for i in range(2):
        sp = _seg_kernel(src3, dst3, g, zerosH)
        g = _stage_call(sp, dinv, params["conv_b"][i], params["bn_g"][i],
                        params["bn_b"][i], params["conv_W"][i + 1])
    sp = _seg_kernel(src3, dst3, g, zerosH)
    og, ol0, ol1, ol2 = _heads_call(sp, dinv, params["conv_b"][2],
                                    params["bn_g"][2], params["bn_b"][2], params)
    return og[:N], ol0[:N], ol1[:N], ol2[:N]
